# baseline (speedup 1.0000x reference)
import jax
import jax.numpy as jnp
from jax.experimental import pallas as pl


_BLOCK_S = 256


def _add_body(x_ref, pos_ref, out_ref):
    out_ref[...] = x_ref[...] + pos_ref[...]


def kernel(x, pos_embedding):
    batch, seq, d = x.shape
    pos = pos_embedding[:seq]
    bs = min(_BLOCK_S, seq)
    grid = (seq // bs, batch)
    return pl.pallas_call(
        _add_body,
        grid=grid,
        in_specs=[
            pl.BlockSpec((1, bs, d), lambda s, b: (b, s, 0)),
            pl.BlockSpec((bs, d), lambda s, b: (s, 0)),
        ],
        out_specs=pl.BlockSpec((1, bs, d), lambda s, b: (b, s, 0)),
        out_shape=jax.ShapeDtypeStruct(x.shape, x.dtype),
    )(x, pos)


# speedup vs baseline: 1.4374x; 1.4374x over previous
import jax
import jax.numpy as jnp
from jax.experimental import pallas as pl


_BLOCK_S = 1024


def _add_body(x_ref, pos_ref, out_ref):
    out_ref[...] = x_ref[...] + pos_ref[...]


def kernel(x, pos_embedding):
    batch, seq, d = x.shape
    pos = pos_embedding[:seq]
    bs = min(_BLOCK_S, seq)
    grid = (seq // bs, batch)
    return pl.pallas_call(
        _add_body,
        grid=grid,
        in_specs=[
            pl.BlockSpec((1, bs, d), lambda s, b: (b, s, 0)),
            pl.BlockSpec((bs, d), lambda s, b: (s, 0)),
        ],
        out_specs=pl.BlockSpec((1, bs, d), lambda s, b: (b, s, 0)),
        out_shape=jax.ShapeDtypeStruct(x.shape, x.dtype),
    )(x, pos)


# speedup vs baseline: 1.5492x; 1.0778x over previous
import jax
import jax.numpy as jnp
from jax.experimental import pallas as pl


_BLOCK_S = 2048


def _add_body(x_ref, pos_ref, out_ref):
    out_ref[...] = x_ref[...] + pos_ref[...]


def kernel(x, pos_embedding):
    batch, seq, d = x.shape
    pos = pos_embedding[:seq]
    bs = min(_BLOCK_S, seq)
    grid = (seq // bs, batch)
    return pl.pallas_call(
        _add_body,
        grid=grid,
        in_specs=[
            pl.BlockSpec((1, bs, d), lambda s, b: (b, s, 0)),
            pl.BlockSpec((bs, d), lambda s, b: (s, 0)),
        ],
        out_specs=pl.BlockSpec((1, bs, d), lambda s, b: (b, s, 0)),
        out_shape=jax.ShapeDtypeStruct(x.shape, x.dtype),
    )(x, pos)
